# Initial kernel scaffold; baseline (speedup 1.0000x reference)
import jax
import jax.numpy as jnp
from jax.experimental import pallas as pl


def kernel(x, edge_index, W, b):
    raise NotImplementedError("write your pallas kernel here")


# speedup vs baseline: 15.4910x; 15.4910x over previous
import functools

import jax
import jax.numpy as jnp
from jax import lax
from jax.experimental import pallas as pl
from jax.experimental.pallas import tpu as pltpu
from jax.experimental.pallas import tpu_sc as plsc

N = 10000
E = 320000
D = 128
NC = 2
NS = 16
DEG_W = 16
CHUNK = 80

E_PER_SC = E // NC
E_PER_TILE = E_PER_SC // NS
N_CHUNKS = E_PER_TILE // CHUNK

_MESH = plsc.VectorSubcoreMesh(
    core_axis_name="c", subcore_axis_name="s", num_cores=NC, num_subcores=NS)


def _strided_copy(n_chunks, sid, body):
  for k in range(-(-n_chunks // NS)):
    j = k * NS + sid

    @pl.when(j < n_chunks)
    def _():
      body(j)


@functools.partial(
    pl.kernel,
    out_type=jax.ShapeDtypeStruct((NC, N, D), jnp.float32),
    mesh=_MESH,
    scratch_types=[
        pltpu.VMEM_SHARED((N, D), jnp.float32),
        pltpu.VMEM((CHUNK,), jnp.int32),
        pltpu.VMEM((CHUNK, D), jnp.float32),
    ],
)
def _deg_kernel(dst_hbm, zrows_hbm, ones_hbm, deg_out, deg_sp, idx_v, ones_v):
  cid = lax.axis_index("c")
  sid = lax.axis_index("s")
  _strided_copy(125, sid,
                lambda j: pltpu.sync_copy(zrows_hbm,
                                          deg_sp.at[pl.ds(j * CHUNK, CHUNK)]))
  pltpu.sync_copy(ones_hbm, ones_v)
  plsc.subcore_barrier()

  base0 = cid * E_PER_SC + sid * E_PER_TILE

  def step(i, _):
    base = base0 + i * CHUNK
    pltpu.sync_copy(dst_hbm.at[pl.ds(base, CHUNK)], idx_v)
    pltpu.sync_copy(ones_v, deg_sp.at[idx_v], add=True)
    return ()

  lax.fori_loop(0, N_CHUNKS, step, ())
  plsc.subcore_barrier()
  _strided_copy(125, sid,
                lambda j: pltpu.sync_copy(
                    deg_sp.at[pl.ds(j * CHUNK, CHUNK)],
                    deg_out.at[cid, pl.ds(j * CHUNK, CHUNK)]))


@functools.partial(
    pl.kernel,
    out_type=jax.ShapeDtypeStruct((NC, N, D), jnp.float32),
    mesh=_MESH,
    scratch_types=[
        pltpu.VMEM_SHARED((N, D), jnp.float32),
        pltpu.VMEM((CHUNK,), jnp.int32),
        pltpu.VMEM((CHUNK,), jnp.int32),
        pltpu.VMEM((CHUNK, D), jnp.float32),
        pltpu.SemaphoreType.DMA,
    ],
)
def _agg_kernel(y_hbm, src_hbm, dst_hbm, zrows_hbm, z_out,
                z_sp, sidx_v, didx_v, rows_v, sem):
  cid = lax.axis_index("c")
  sid = lax.axis_index("s")
  _strided_copy(125, sid,
                lambda j: pltpu.sync_copy(zrows_hbm,
                                          z_sp.at[pl.ds(j * CHUNK, CHUNK)]))
  plsc.subcore_barrier()

  base0 = cid * E_PER_SC + sid * E_PER_TILE

  def step(i, _):
    base = base0 + i * CHUNK
    pltpu.sync_copy(src_hbm.at[pl.ds(base, CHUNK)], sidx_v)
    pltpu.sync_copy(dst_hbm.at[pl.ds(base, CHUNK)], didx_v)
    pltpu.async_copy(y_hbm.at[sidx_v], rows_v, sem).wait()
    pltpu.sync_copy(rows_v, z_sp.at[didx_v], add=True)
    return ()

  lax.fori_loop(0, N_CHUNKS, step, ())
  plsc.subcore_barrier()
  _strided_copy(125, sid,
                lambda j: pltpu.sync_copy(
                    z_sp.at[pl.ds(j * CHUNK, CHUNK)],
                    z_out.at[cid, pl.ds(j * CHUNK, CHUNK)]))


_ROWS_TC = 1000


def _xw_body(deg_ref, x_ref, w_ref, o_ref):
  deg = deg_ref[0, :, 0:1] + deg_ref[1, :, 0:1] + 1.0
  dis = lax.rsqrt(deg)
  o_ref[...] = dis * jnp.dot(x_ref[...], w_ref[...],
                             preferred_element_type=jnp.float32)


def _final_body(z_ref, y_ref, deg_ref, b_ref, o_ref):
  deg = deg_ref[0, :, 0:1] + deg_ref[1, :, 0:1] + 1.0
  dis = lax.rsqrt(deg)
  o_ref[...] = dis * (z_ref[0] + z_ref[1] + y_ref[...]) + b_ref[...]


def _tc_xw(degparts, x, W):
  grid = (N // _ROWS_TC,)
  return pl.pallas_call(
      _xw_body,
      grid=grid,
      in_specs=[
          pl.BlockSpec((NC, _ROWS_TC, D), lambda i: (0, i, 0)),
          pl.BlockSpec((_ROWS_TC, D), lambda i: (i, 0)),
          pl.BlockSpec((D, D), lambda i: (0, 0)),
      ],
      out_specs=pl.BlockSpec((_ROWS_TC, D), lambda i: (i, 0)),
      out_shape=jax.ShapeDtypeStruct((N, D), jnp.float32),
  )(degparts, x, W)


def _tc_final(zparts, y, degparts, b2):
  grid = (N // _ROWS_TC,)
  return pl.pallas_call(
      _final_body,
      grid=grid,
      in_specs=[
          pl.BlockSpec((NC, _ROWS_TC, D), lambda i: (0, i, 0)),
          pl.BlockSpec((_ROWS_TC, D), lambda i: (i, 0)),
          pl.BlockSpec((NC, _ROWS_TC, D), lambda i: (0, i, 0)),
          pl.BlockSpec((1, D), lambda i: (0, 0)),
      ],
      out_specs=pl.BlockSpec((_ROWS_TC, D), lambda i: (i, 0)),
      out_shape=jax.ShapeDtypeStruct((N, D), jnp.float32),
  )(zparts, y, degparts, b2)


def kernel(x, edge_index, W, b):
  src = edge_index[0].astype(jnp.int32)
  dst = edge_index[1].astype(jnp.int32)
  ones = jnp.ones((CHUNK, D), jnp.float32)
  zrows = jnp.zeros((CHUNK, D), jnp.float32)

  degparts = _deg_kernel(dst, zrows, ones)
  y = _tc_xw(degparts, x, W)
  zparts = _agg_kernel(y, src, dst, zrows)
  return _tc_final(zparts, y, degparts, b.reshape(1, D))


# speedup vs baseline: 26.3640x; 1.7019x over previous
import functools

import jax
import jax.numpy as jnp
from jax import lax
from jax.experimental import pallas as pl
from jax.experimental.pallas import tpu as pltpu
from jax.experimental.pallas import tpu_sc as plsc

N = 10000
E = 320000
D = 128
NC = 2
NS = 16
NW = NC * NS
CHUNK = 80
N_CHUNKS = (E // NW) // CHUNK

_MESH = plsc.VectorSubcoreMesh(
    core_axis_name="c", subcore_axis_name="s", num_cores=NC, num_subcores=NS)


def _strided_copy(n_chunks, sid, body):
  for k in range(-(-n_chunks // NS)):
    j = k * NS + sid

    @pl.when(j < n_chunks)
    def _():
      body(j)


@functools.partial(
    pl.kernel,
    out_type=jax.ShapeDtypeStruct((NC, N, D), jnp.float32),
    mesh=_MESH,
    scratch_types=[
        pltpu.VMEM_SHARED((N, D), jnp.float32),
        pltpu.VMEM((N_CHUNKS, CHUNK), jnp.int32),
        pltpu.VMEM((CHUNK, D), jnp.float32),
        pltpu.SemaphoreType.DMA,
    ],
)
def _deg_kernel(dst3d_hbm, zrows_hbm, ones_hbm, deg_out,
                deg_sp, didx_all, ones_v, sem):
  cid = lax.axis_index("c")
  sid = lax.axis_index("s")
  _strided_copy(N_CHUNKS, sid,
                lambda j: pltpu.sync_copy(zrows_hbm,
                                          deg_sp.at[pl.ds(j * CHUNK, CHUNK)]))
  pltpu.sync_copy(ones_hbm, ones_v)
  wid = cid * NS + sid
  pltpu.sync_copy(dst3d_hbm.at[wid], didx_all)
  plsc.subcore_barrier()

  FD = 5

  def blk(k, _):
    base = k * FD
    for j in range(FD):
      pltpu.async_copy(ones_v, deg_sp.at[didx_all.at[base + j]], sem, add=True)
    for j in range(FD):
      pltpu.make_async_copy(ones_v, deg_sp.at[didx_all.at[base + j]],
                            sem).wait()
    return ()

  lax.fori_loop(0, N_CHUNKS // FD, blk, ())
  plsc.subcore_barrier()
  _strided_copy(N_CHUNKS, sid,
                lambda j: pltpu.sync_copy(
                    deg_sp.at[pl.ds(j * CHUNK, CHUNK)],
                    deg_out.at[cid, pl.ds(j * CHUNK, CHUNK)]))


@functools.partial(
    pl.kernel,
    out_type=jax.ShapeDtypeStruct((NC, N, D), jnp.float32),
    mesh=_MESH,
    scratch_types=[
        pltpu.VMEM_SHARED((N, D), jnp.float32),
        [pltpu.VMEM((CHUNK,), jnp.int32)] * 2,
        [pltpu.VMEM((CHUNK,), jnp.int32)] * 2,
        [pltpu.VMEM((CHUNK, D), jnp.float32)] * 2,
        [pltpu.SemaphoreType.DMA] * 2,
        [pltpu.SemaphoreType.DMA] * 2,
    ],
)
def _agg_kernel(y_hbm, src_hbm, dst_hbm, zrows_hbm, z_out,
                z_sp, sidx, didx, rows, semi, semg):
  cid = lax.axis_index("c")
  sid = lax.axis_index("s")
  _strided_copy(N_CHUNKS, sid,
                lambda j: pltpu.sync_copy(zrows_hbm,
                                          z_sp.at[pl.ds(j * CHUNK, CHUNK)]))
  wid = cid * NS + sid
  base0 = wid * N_CHUNKS

  def load_idx(j, s, sync):
    hs = src_hbm.at[pl.ds((base0 + j) * CHUNK, CHUNK)]
    hd = dst_hbm.at[pl.ds((base0 + j) * CHUNK, CHUNK)]
    if sync:
      pltpu.sync_copy(hs, sidx[s])
      pltpu.sync_copy(hd, didx[s])
    else:
      pltpu.async_copy(hs, sidx[s], semi[s])
      pltpu.async_copy(hd, didx[s], semi[s])

  def wait_idx(j, s):
    eb = (base0 + j) * CHUNK
    pltpu.make_async_copy(src_hbm.at[pl.ds(eb, CHUNK)], sidx[s],
                          semi[s]).wait()
    pltpu.make_async_copy(dst_hbm.at[pl.ds(eb, CHUNK)], didx[s],
                          semi[s]).wait()

  plsc.subcore_barrier()

  load_idx(0, 0, True)
  pltpu.async_copy(y_hbm.at[sidx[0]], rows[0], semg[0])
  load_idx(1, 1, False)

  def handle(j, cur, nxt):
    @pl.when(j + 1 < N_CHUNKS)
    def _():
      wait_idx(j + 1, nxt)
      pltpu.async_copy(y_hbm.at[sidx[nxt]], rows[nxt], semg[nxt])

    pltpu.make_async_copy(y_hbm.at[sidx[cur]], rows[cur], semg[cur]).wait()
    pltpu.sync_copy(rows[cur], z_sp.at[didx[cur]], add=True)

    @pl.when(j + 2 < N_CHUNKS)
    def _():
      load_idx(j + 2, cur, False)

  def step(k, _):
    a = 2 * k
    handle(a, 0, 1)

    @pl.when(a + 1 < N_CHUNKS)
    def _():
      handle(a + 1, 1, 0)

    return ()

  lax.fori_loop(0, (N_CHUNKS + 1) // 2, step, ())
  plsc.subcore_barrier()
  _strided_copy(N_CHUNKS, sid,
                lambda j: pltpu.sync_copy(
                    z_sp.at[pl.ds(j * CHUNK, CHUNK)],
                    z_out.at[cid, pl.ds(j * CHUNK, CHUNK)]))


_ROWS_TC = 1000


def _xw_body(deg_ref, x_ref, w_ref, o_ref):
  deg = deg_ref[0, :, 0:1] + deg_ref[1, :, 0:1] + 1.0
  dis = lax.rsqrt(deg)
  o_ref[...] = dis * jnp.dot(x_ref[...], w_ref[...],
                             preferred_element_type=jnp.float32)


def _final_body(z_ref, y_ref, deg_ref, b_ref, o_ref):
  deg = deg_ref[0, :, 0:1] + deg_ref[1, :, 0:1] + 1.0
  dis = lax.rsqrt(deg)
  o_ref[...] = dis * (z_ref[0] + z_ref[1] + y_ref[...]) + b_ref[...]


def _tc_xw(degparts, x, W):
  grid = (N // _ROWS_TC,)
  return pl.pallas_call(
      _xw_body,
      grid=grid,
      in_specs=[
          pl.BlockSpec((NC, _ROWS_TC, D), lambda i: (0, i, 0)),
          pl.BlockSpec((_ROWS_TC, D), lambda i: (i, 0)),
          pl.BlockSpec((D, D), lambda i: (0, 0)),
      ],
      out_specs=pl.BlockSpec((_ROWS_TC, D), lambda i: (i, 0)),
      out_shape=jax.ShapeDtypeStruct((N, D), jnp.float32),
  )(degparts, x, W)


def _tc_final(zparts, y, degparts, b2):
  grid = (N // _ROWS_TC,)
  return pl.pallas_call(
      _final_body,
      grid=grid,
      in_specs=[
          pl.BlockSpec((NC, _ROWS_TC, D), lambda i: (0, i, 0)),
          pl.BlockSpec((_ROWS_TC, D), lambda i: (i, 0)),
          pl.BlockSpec((NC, _ROWS_TC, D), lambda i: (0, i, 0)),
          pl.BlockSpec((1, D), lambda i: (0, 0)),
      ],
      out_specs=pl.BlockSpec((_ROWS_TC, D), lambda i: (i, 0)),
      out_shape=jax.ShapeDtypeStruct((N, D), jnp.float32),
  )(zparts, y, degparts, b2)


def kernel(x, edge_index, W, b):
  src = edge_index[0].astype(jnp.int32)
  dst = edge_index[1].astype(jnp.int32)
  dst3d = dst.reshape(NW, N_CHUNKS, CHUNK)
  ones = jnp.ones((CHUNK, D), jnp.float32)
  zrows = jnp.zeros((CHUNK, D), jnp.float32)

  degparts = _deg_kernel(dst3d, zrows, ones)
  y = _tc_xw(degparts, x, W)
  zparts = _agg_kernel(y, src, dst, zrows)
  return _tc_final(zparts, y, degparts, b.reshape(1, D))


# speedup vs baseline: 28.0879x; 1.0654x over previous
import functools

import jax
import jax.numpy as jnp
from jax import lax
from jax.experimental import pallas as pl
from jax.experimental.pallas import tpu as pltpu
from jax.experimental.pallas import tpu_sc as plsc

N = 10000
E = 320000
D = 128
NC = 2
NS = 16
NW = NC * NS
CHUNK = 80
N_CHUNKS = (E // NW) // CHUNK
RING = 3

_MESH = plsc.VectorSubcoreMesh(
    core_axis_name="c", subcore_axis_name="s", num_cores=NC, num_subcores=NS)


def _strided_copy(n_chunks, sid, body):
  for k in range(-(-n_chunks // NS)):
    j = k * NS + sid

    @pl.when(j < n_chunks)
    def _():
      body(j)


@functools.partial(
    pl.kernel,
    out_type=jax.ShapeDtypeStruct((NC, N, D), jnp.float32),
    mesh=_MESH,
    scratch_types=[
        pltpu.VMEM_SHARED((N, D), jnp.float32),
        pltpu.VMEM((N_CHUNKS, CHUNK), jnp.int32),
        pltpu.VMEM((CHUNK, D), jnp.float32),
        pltpu.SemaphoreType.DMA,
    ],
)
def _deg_kernel(dst3d_hbm, zrows_hbm, ones_hbm, deg_out,
                deg_sp, didx_all, ones_v, sem):
  cid = lax.axis_index("c")
  sid = lax.axis_index("s")
  _strided_copy(N_CHUNKS, sid,
                lambda j: pltpu.sync_copy(zrows_hbm,
                                          deg_sp.at[pl.ds(j * CHUNK, CHUNK)]))
  pltpu.sync_copy(ones_hbm, ones_v)
  wid = cid * NS + sid
  pltpu.sync_copy(dst3d_hbm.at[wid], didx_all)
  plsc.subcore_barrier()

  FD = 5

  def blk(k, _):
    base = k * FD
    for j in range(FD):
      pltpu.async_copy(ones_v, deg_sp.at[didx_all.at[base + j]], sem, add=True)
    for j in range(FD):
      pltpu.make_async_copy(ones_v, deg_sp.at[didx_all.at[base + j]],
                            sem).wait()
    return ()

  lax.fori_loop(0, N_CHUNKS // FD, blk, ())
  plsc.subcore_barrier()
  _strided_copy(N_CHUNKS, sid,
                lambda j: pltpu.sync_copy(
                    deg_sp.at[pl.ds(j * CHUNK, CHUNK)],
                    deg_out.at[cid, pl.ds(j * CHUNK, CHUNK)]))


@functools.partial(
    pl.kernel,
    out_type=jax.ShapeDtypeStruct((NC, N, D), jnp.float32),
    mesh=_MESH,
    scratch_types=[
        pltpu.VMEM_SHARED((N, D), jnp.float32),
        [pltpu.VMEM((CHUNK,), jnp.int32)] * RING,
        [pltpu.VMEM((CHUNK,), jnp.int32)] * RING,
        [pltpu.VMEM((CHUNK, D), jnp.float32)] * RING,
        [pltpu.SemaphoreType.DMA] * RING,
        [pltpu.SemaphoreType.DMA] * RING,
        pltpu.SemaphoreType.DMA,
    ],
)
def _agg_kernel(y_hbm, src_hbm, dst_hbm, zrows_hbm, z_out,
                z_sp, sidx, didx, rows, semi, semg, sems):
  cid = lax.axis_index("c")
  sid = lax.axis_index("s")
  _strided_copy(N_CHUNKS, sid,
                lambda j: pltpu.sync_copy(zrows_hbm,
                                          z_sp.at[pl.ds(j * CHUNK, CHUNK)]))
  wid = cid * NS + sid
  base0 = wid * N_CHUNKS

  def load_idx(j, s, sync):
    hs = src_hbm.at[pl.ds((base0 + j) * CHUNK, CHUNK)]
    hd = dst_hbm.at[pl.ds((base0 + j) * CHUNK, CHUNK)]
    if sync:
      pltpu.sync_copy(hs, sidx[s])
      pltpu.sync_copy(hd, didx[s])
    else:
      pltpu.async_copy(hs, sidx[s], semi[s])
      pltpu.async_copy(hd, didx[s], semi[s])

  def wait_idx(j, s):
    eb = (base0 + j) * CHUNK
    pltpu.make_async_copy(src_hbm.at[pl.ds(eb, CHUNK)], sidx[s],
                          semi[s]).wait()
    pltpu.make_async_copy(dst_hbm.at[pl.ds(eb, CHUNK)], didx[s],
                          semi[s]).wait()

  def drain_scatter():
    pltpu.make_async_copy(rows[0], z_sp.at[didx[0]], sems).wait()

  plsc.subcore_barrier()

  load_idx(0, 0, True)
  pltpu.async_copy(y_hbm.at[sidx[0]], rows[0], semg[0])
  load_idx(1, 1, False)

  def handle(j, s, drain):
    nxt = (s + 1) % RING

    @pl.when(j + 1 < N_CHUNKS)
    def _():
      wait_idx(j + 1, nxt)
      pltpu.async_copy(y_hbm.at[sidx[nxt]], rows[nxt], semg[nxt])

    pltpu.make_async_copy(y_hbm.at[sidx[s]], rows[s], semg[s]).wait()
    pltpu.async_copy(rows[s], z_sp.at[didx[s]], sems, add=True)
    if drain:
      drain_scatter()

    @pl.when(j + 2 < N_CHUNKS)
    def _():
      load_idx(j + 2, (s + 2) % RING, False)

  def step(k, _):
    base = 3 * k
    for u in range(3):
      handle(base + u, u, True)
    return ()

  handle(0, 0, False)
  handle(1, 1, True)
  handle(2, 2, True)
  lax.fori_loop(1, N_CHUNKS // 3, step, ())
  handle(N_CHUNKS - 2, 0, True)
  handle(N_CHUNKS - 1, 1, True)
  drain_scatter()
  plsc.subcore_barrier()
  _strided_copy(N_CHUNKS, sid,
                lambda j: pltpu.sync_copy(
                    z_sp.at[pl.ds(j * CHUNK, CHUNK)],
                    z_out.at[cid, pl.ds(j * CHUNK, CHUNK)]))


_ROWS_TC = 1000


def _xw_body(x_ref, w_ref, o_ref):
  o_ref[...] = jnp.dot(x_ref[...], w_ref[...],
                       preferred_element_type=jnp.float32)


def _scale_body(deg_ref, xw_ref, y_ref, dis_ref):
  deg = deg_ref[0, :, 0:1] + deg_ref[1, :, 0:1] + 1.0
  dis = lax.rsqrt(deg)
  dis_ref[...] = dis
  y_ref[...] = dis * xw_ref[...]


def _final_body(z_ref, y_ref, dis_ref, b_ref, o_ref):
  o_ref[...] = dis_ref[...] * (z_ref[0] + z_ref[1] + y_ref[...]) + b_ref[...]


def _tc_xw(x, W):
  grid = (N // _ROWS_TC,)
  return pl.pallas_call(
      _xw_body,
      grid=grid,
      in_specs=[
          pl.BlockSpec((_ROWS_TC, D), lambda i: (i, 0)),
          pl.BlockSpec((D, D), lambda i: (0, 0)),
      ],
      out_specs=pl.BlockSpec((_ROWS_TC, D), lambda i: (i, 0)),
      out_shape=jax.ShapeDtypeStruct((N, D), jnp.float32),
  )(x, W)


def _tc_scale(degparts, xw):
  grid = (N // _ROWS_TC,)
  return pl.pallas_call(
      _scale_body,
      grid=grid,
      in_specs=[
          pl.BlockSpec((NC, _ROWS_TC, D), lambda i: (0, i, 0)),
          pl.BlockSpec((_ROWS_TC, D), lambda i: (i, 0)),
      ],
      out_specs=[
          pl.BlockSpec((_ROWS_TC, D), lambda i: (i, 0)),
          pl.BlockSpec((_ROWS_TC, 1), lambda i: (i, 0)),
      ],
      out_shape=[
          jax.ShapeDtypeStruct((N, D), jnp.float32),
          jax.ShapeDtypeStruct((N, 1), jnp.float32),
      ],
  )(degparts, xw)


def _tc_final(zparts, y, dis, b2):
  grid = (N // _ROWS_TC,)
  return pl.pallas_call(
      _final_body,
      grid=grid,
      in_specs=[
          pl.BlockSpec((NC, _ROWS_TC, D), lambda i: (0, i, 0)),
          pl.BlockSpec((_ROWS_TC, D), lambda i: (i, 0)),
          pl.BlockSpec((_ROWS_TC, 1), lambda i: (i, 0)),
          pl.BlockSpec((1, D), lambda i: (0, 0)),
      ],
      out_specs=pl.BlockSpec((_ROWS_TC, D), lambda i: (i, 0)),
      out_shape=jax.ShapeDtypeStruct((N, D), jnp.float32),
  )(zparts, y, dis, b2)


def kernel(x, edge_index, W, b):
  src = edge_index[0].astype(jnp.int32)
  dst = edge_index[1].astype(jnp.int32)
  dst3d = dst.reshape(NW, N_CHUNKS, CHUNK)
  ones = jnp.ones((CHUNK, D), jnp.float32)
  zrows = jnp.zeros((CHUNK, D), jnp.float32)

  xw = _tc_xw(x, W)
  degparts = _deg_kernel(dst3d, zrows, ones)
  y, dis = _tc_scale(degparts, xw)
  zparts = _agg_kernel(y, src, dst, zrows)
  return _tc_final(zparts, y, dis, b.reshape(1, D))
